# baseline (speedup 1.0000x reference)
import jax
import jax.numpy as jnp
from jax.experimental import pallas as pl
from jax.experimental.pallas import tpu as pltpu

N = 10000
F_IN = 128
F_OUT = 128
BM = 200
HALF = N // 2
NB = HALF // BM


def _body(adj_top_ref, adj_bot_ref, feat_ref, w_ref, out_ref, fe_ref):
    @pl.when(pl.program_id(0) == 0)
    def _():
        fe_ref[...] = jnp.dot(feat_ref[...], w_ref[...],
                              preferred_element_type=jnp.float32)

    fe = fe_ref[...]
    out_ref[0] = jnp.dot(adj_top_ref[...], fe,
                         preferred_element_type=jnp.float32)
    out_ref[1] = jnp.dot(adj_bot_ref[...], fe,
                         preferred_element_type=jnp.float32)


def kernel(feat, adj, weight):
    out = pl.pallas_call(
        _body,
        grid=(NB,),
        in_specs=[
            pl.BlockSpec((BM, N), lambda i: (i, 0)),
            pl.BlockSpec((BM, N), lambda i: (i + NB, 0)),
            pl.BlockSpec((N, F_IN), lambda i: (0, 0)),
            pl.BlockSpec((F_IN, F_OUT), lambda i: (0, 0)),
        ],
        out_specs=pl.BlockSpec((2, BM, F_OUT), lambda i: (0, i, 0)),
        out_shape=jax.ShapeDtypeStruct((2, HALF, F_OUT), jnp.float32),
        scratch_shapes=[pltpu.VMEM((N, F_OUT), jnp.float32)],
    )(adj, adj, feat, weight)
    return out.reshape(N, F_OUT)
